# baseline (speedup 1.0000x reference)
import functools

import jax
import jax.numpy as jnp
from jax import lax
from jax.experimental import pallas as pl
from jax.experimental.pallas import tpu as pltpu
from jax.experimental.pallas import tpu_sc as plsc

NC = 2
NS = 16
NW = NC * NS
NB = 4
PF = 2
NSPLIT = 2


def _make_half(B, TH, D):
    nchunks = B // NW

    mesh = plsc.VectorSubcoreMesh(
        core_axis_name="c", subcore_axis_name="s", num_cores=NC, num_subcores=NS
    )

    @functools.partial(
        pl.kernel,
        out_type=jax.ShapeDtypeStruct((B, TH, D), jnp.float32),
        mesh=mesh,
        compiler_params=pltpu.CompilerParams(use_tc_tiling_on_sc=False),
        scratch_types=[
            pltpu.VMEM((nchunks, TH), jnp.int32),
            pltpu.VMEM((TH, D), jnp.float32),
        ]
        + [pltpu.VMEM((TH, D), jnp.float32) for _ in range(NB)]
        + [pltpu.SemaphoreType.DMA for _ in range(2 * NB)],
    )
    def run(x_hbm, tok_hbm, pos_hbm, out_hbm, idx_v, pos_v, *refs):
        bufs = refs[:NB]
        gsem = refs[NB:2 * NB]
        ssem = refs[2 * NB:3 * NB]
        wid = lax.axis_index("s") * NC + lax.axis_index("c")
        seq0 = wid * nchunks
        pltpu.sync_copy(x_hbm.at[pl.ds(seq0, nchunks)], idx_v)
        pltpu.sync_copy(pos_hbm, pos_v)

        def gstart(j, b):
            pltpu.async_copy(tok_hbm.at[idx_v.at[j]], bufs[b], gsem[b])

        def gwait(j, b):
            pltpu.make_async_copy(tok_hbm.at[idx_v.at[j]], bufs[b], gsem[b]).wait()

        def sstart(j, b):
            pltpu.async_copy(bufs[b], out_hbm.at[seq0 + j], ssem[b])

        def swait(b):
            pltpu.make_async_copy(bufs[b], out_hbm.at[0], ssem[b]).wait()

        RU = 10

        def add_pos(b):
            def rows(r2, c2):
                for rr in range(RU):
                    row = RU * r2 + rr
                    for c in range(D // 16):
                        sl = pl.ds(c * 16, 16)
                        plsc.addupdate(bufs[b].at[row, sl], pos_v[row, sl])
                return c2

            lax.fori_loop(0, TH // RU, rows, 0)

        def consume(j, b):
            gwait(j, b)
            add_pos(b)
            sstart(j, b)

        gstart(0, 0)
        gstart(1, 1)
        consume(0, 0)
        gstart(2, 2)
        consume(1, 1)
        gstart(3, 3)
        consume(2, 2)
        swait(0)
        gstart(4, 0)
        consume(3, 3)
        swait(1)
        gstart(5, 1)

        def group(g, c2):
            j0 = NB * g
            for r in range(NB):
                j = j0 + r
                bp = (r + PF) % NB
                consume(j, r)
                swait(bp)
                gstart(j + PF, bp)
            return c2

        lax.fori_loop(1, nchunks // NB - 1, group, 0)

        j0 = nchunks - NB
        consume(j0, 0)
        swait(2)
        gstart(j0 + 2, 2)
        consume(j0 + 1, 1)
        swait(3)
        gstart(j0 + 3, 3)
        consume(j0 + 2, 2)
        consume(j0 + 3, 3)
        for b in range(NB):
            swait(b)

    return run


def kernel(x, token_table, pos_table):
    B, T = x.shape
    D = token_table.shape[1]
    TH = T // NSPLIT
    run = _make_half(B, TH, D)
    outs = []
    for h in range(NSPLIT):
        xh = lax.slice_in_dim(x, h * TH, (h + 1) * TH, axis=1)
        ph = lax.slice_in_dim(pos_table, h * TH, (h + 1) * TH, axis=0)
        outs.append(run(xh, token_table, ph))
    return jnp.concatenate(outs, axis=1)


# speedup vs baseline: 1.2125x; 1.2125x over previous
import functools

import jax
import jax.numpy as jnp
from jax import lax
from jax.experimental import pallas as pl
from jax.experimental.pallas import tpu as pltpu
from jax.experimental.pallas import tpu_sc as plsc

NC = 2
NS = 16
NW = NC * NS
SPLITS = ((0, 96), (96, 104))
NB = 4
PF = 2


def kernel(x, token_table, pos_table):
    B, T = x.shape
    D = token_table.shape[1]
    nchunks = B // NW

    mesh = plsc.VectorSubcoreMesh(
        core_axis_name="c", subcore_axis_name="s", num_cores=NC, num_subcores=NS
    )

    @functools.partial(
        pl.kernel,
        out_type=jax.ShapeDtypeStruct((B, T, D), jnp.float32),
        mesh=mesh,
        compiler_params=pltpu.CompilerParams(use_tc_tiling_on_sc=False),
        scratch_types=[
            pltpu.VMEM((nchunks, T), jnp.int32),
            pltpu.VMEM((T, D), jnp.float32),
        ]
        + [pltpu.VMEM((T, D), jnp.float32) for _ in range(NB)]
        + [pltpu.SemaphoreType.DMA for _ in range(2 * NB)],
    )
    def run(x_hbm, tok_hbm, pos_hbm, out_hbm, idx_v, pos_v, *bufs_and_sems):
        bufs = bufs_and_sems[:NB]
        gsem = bufs_and_sems[NB:2 * NB]
        ssem = bufs_and_sems[2 * NB:3 * NB]
        wid = lax.axis_index("s") * NC + lax.axis_index("c")
        seq0 = wid * nchunks
        pltpu.sync_copy(x_hbm.at[pl.ds(seq0, nchunks)], idx_v)
        pltpu.sync_copy(pos_hbm, pos_v)

        def gstart(j, b):
            for off, n in SPLITS:
                pltpu.async_copy(
                    tok_hbm.at[idx_v.at[j, pl.ds(off, n)]],
                    bufs[b].at[pl.ds(off, n)],
                    gsem[b],
                )

        def gwait(j, b):
            for off, n in SPLITS:
                pltpu.make_async_copy(
                    tok_hbm.at[idx_v.at[j, pl.ds(off, n)]],
                    bufs[b].at[pl.ds(off, n)],
                    gsem[b],
                ).wait()

        def sstart(j, b):
            pltpu.async_copy(bufs[b], out_hbm.at[seq0 + j], ssem[b])

        def swait(b):
            pltpu.make_async_copy(bufs[b], out_hbm.at[0], ssem[b]).wait()

        RU = 8

        def add_pos(b):
            def rows(r2, c2):
                for rr in range(RU):
                    row = RU * r2 + rr
                    for c in range(D // 16):
                        sl = pl.ds(c * 16, 16)
                        plsc.addupdate(bufs[b].at[row, sl], pos_v[row, sl])
                return c2

            lax.fori_loop(0, T // RU, rows, 0)

        def consume(j, b):
            gwait(j, b)
            add_pos(b)
            sstart(j, b)

        gstart(0, 0)
        gstart(1, 1)
        consume(0, 0)
        gstart(2, 2)
        consume(1, 1)
        gstart(3, 3)
        consume(2, 2)
        swait(0)
        gstart(4, 0)
        consume(3, 3)
        swait(1)
        gstart(5, 1)

        def group(g, c2):
            j0 = NB * g
            for r in range(NB):
                j = j0 + r
                bp = (r + PF) % NB
                consume(j, r)
                swait(bp)
                gstart(j + PF, bp)
            return c2

        lax.fori_loop(1, nchunks // NB - 1, group, 0)

        j0 = nchunks - NB
        consume(j0, 0)
        swait(2)
        gstart(j0 + 2, 2)
        consume(j0 + 1, 1)
        swait(3)
        gstart(j0 + 3, 3)
        consume(j0 + 2, 2)
        consume(j0 + 3, 3)
        for b in range(NB):
            swait(b)

    return run(x, token_table, pos_table)


# speedup vs baseline: 1.6033x; 1.3223x over previous
import functools

import jax
import jax.numpy as jnp
from jax import lax
from jax.experimental import pallas as pl
from jax.experimental.pallas import tpu as pltpu
from jax.experimental.pallas import tpu_sc as plsc

NC = 2
NS = 16
NW = NC * NS
SPLITS = ((0, 96), (96, 104))
NB = 4
PF = 2


def kernel(x, token_table, pos_table):
    B, T = x.shape
    D = token_table.shape[1]
    nchunks = B // NW

    mesh = plsc.VectorSubcoreMesh(
        core_axis_name="c", subcore_axis_name="s", num_cores=NC, num_subcores=NS
    )

    @functools.partial(
        pl.kernel,
        out_type=jax.ShapeDtypeStruct((B, T, 2 * D), jnp.float32),
        mesh=mesh,
        compiler_params=pltpu.CompilerParams(use_tc_tiling_on_sc=False),
        scratch_types=[
            pltpu.VMEM((nchunks, T), jnp.int32),
            pltpu.VMEM((T, D), jnp.float32),
        ]
        + [pltpu.VMEM((T, D), jnp.float32) for _ in range(NB)]
        + [pltpu.SemaphoreType.DMA for _ in range(2 * NB)],
    )
    def run(x_hbm, tok_hbm, pos_hbm, out_hbm, idx_v, pos_v, *bufs_and_sems):
        bufs = bufs_and_sems[:NB]
        gsem = bufs_and_sems[NB:2 * NB]
        ssem = bufs_and_sems[2 * NB:3 * NB]
        wid = lax.axis_index("s") * NC + lax.axis_index("c")
        seq0 = wid * nchunks
        pltpu.sync_copy(x_hbm.at[pl.ds(seq0, nchunks)], idx_v)
        pltpu.sync_copy(pos_hbm, pos_v)

        def gstart(j, b):
            for off, n in SPLITS:
                pltpu.async_copy(
                    tok_hbm.at[idx_v.at[j, pl.ds(off, n)]],
                    bufs[b].at[pl.ds(off, n)],
                    gsem[b],
                )

        def gwait(j, b):
            for off, n in SPLITS:
                pltpu.make_async_copy(
                    tok_hbm.at[idx_v.at[j, pl.ds(off, n)]],
                    bufs[b].at[pl.ds(off, n)],
                    gsem[b],
                ).wait()

        def sstart(j, b):
            pltpu.async_copy(
                bufs[b], out_hbm.at[seq0 + j, :, pl.ds(0, D)], ssem[b]
            )

        def swait(b):
            pltpu.make_async_copy(
                bufs[b], out_hbm.at[0, :, pl.ds(0, D)], ssem[b]
            ).wait()

        RU = 8

        def add_pos(b):
            def rows(r2, c2):
                for rr in range(RU):
                    row = RU * r2 + rr
                    for c in range(D // 16):
                        sl = pl.ds(c * 16, 16)
                        plsc.addupdate(bufs[b].at[row, sl], pos_v[row, sl])
                return c2

            lax.fori_loop(0, T // RU, rows, 0)

        def consume(j, b):
            gwait(j, b)
            add_pos(b)
            sstart(j, b)

        gstart(0, 0)
        gstart(1, 1)
        consume(0, 0)
        gstart(2, 2)
        consume(1, 1)
        gstart(3, 3)
        consume(2, 2)
        swait(0)
        gstart(4, 0)
        consume(3, 3)
        swait(1)
        gstart(5, 1)

        def group(g, c2):
            j0 = NB * g
            for r in range(NB):
                j = j0 + r
                bp = (r + PF) % NB
                consume(j, r)
                swait(bp)
                gstart(j + PF, bp)
            return c2

        lax.fori_loop(1, nchunks // NB - 1, group, 0)

        j0 = nchunks - NB
        consume(j0, 0)
        swait(2)
        gstart(j0 + 2, 2)
        consume(j0 + 1, 1)
        swait(3)
        gstart(j0 + 3, 3)
        consume(j0 + 2, 2)
        consume(j0 + 3, 3)
        for b in range(NB):
            swait(b)

    padded = run(x, token_table, pos_table)
    return padded[:, :, :D]
